# Initial kernel scaffold; baseline (speedup 1.0000x reference)
import jax
import jax.numpy as jnp
from jax.experimental import pallas as pl


def kernel(pos, edge_index, edge_attr, batch_index, params):
    raise NotImplementedError("write your pallas kernel here")


# speedup vs baseline: 1.5374x; 1.5374x over previous
import functools

import jax
import jax.numpy as jnp
from jax import lax
from jax.experimental import pallas as pl
from jax.experimental.pallas import tpu as pltpu
from jax.experimental.pallas import tpu_sc as plsc

N_NODES = 10000
N_PAD = 10240
N_EDGES = 160000
N_GRAPHS = 16
HIDDEN = 16
EDGE_W = 64

NC = 2
NS = 16
NW = NC * NS

_MESH = plsc.VectorSubcoreMesh(core_axis_name="c", subcore_axis_name="s")

GATHER_CHUNK = 2000


def _gather_body(x_hbm, idx_hbm, out_hbm, idx_v, rows_v, x_sh, sem):
    wid = lax.axis_index("s") * NC + lax.axis_index("c")
    s = lax.axis_index("s")
    row0 = s * ROWS_PER_TILE
    pltpu.sync_copy(x_hbm.at[pl.ds(row0, ROWS_PER_TILE)],
                    x_sh.at[pl.ds(row0, ROWS_PER_TILE)])
    plsc.subcore_barrier()
    base = wid * ((2 * N_EDGES) // NW)

    def body(ci, _):
        off = base + ci * GATHER_CHUNK
        pltpu.sync_copy(idx_hbm.at[pl.ds(off, GATHER_CHUNK)], idx_v)
        pltpu.async_copy(x_sh.at[idx_v], rows_v, sem).wait()
        pltpu.sync_copy(rows_v, out_hbm.at[pl.ds(off, GATHER_CHUNK)])
        return ()

    lax.fori_loop(0, (2 * N_EDGES) // NW // GATHER_CHUNK, body, ())


def _sc_gather(x, idx_all):
    k = pl.kernel(
        _gather_body,
        out_type=jax.ShapeDtypeStruct((2 * N_EDGES, HIDDEN), jnp.float32),
        mesh=_MESH,
        compiler_params=pltpu.CompilerParams(use_tc_tiling_on_sc=False),
        scratch_types=[
            pltpu.VMEM((GATHER_CHUNK,), jnp.int32),
            pltpu.VMEM((GATHER_CHUNK, HIDDEN), jnp.float32),
            pltpu.VMEM_SHARED((N_PAD, HIDDEN), jnp.float32),
            pltpu.SemaphoreType.DMA,
        ],
    )
    return k(x, idx_all)


SCAT_CHUNK = 200
ROWS_PER_TILE = N_PAD // NS


def _scatter_body(msg_hbm, dst_hbm, zeros_hbm, out_hbm, idx_v, msg_v,
                  accum_sh, sem):
    c = lax.axis_index("c")
    s = lax.axis_index("s")
    wid = s * NC + c
    base = wid * (N_EDGES // NW)
    row0 = s * ROWS_PER_TILE
    pltpu.sync_copy(zeros_hbm.at[pl.ds(row0, ROWS_PER_TILE)],
                    accum_sh.at[pl.ds(row0, ROWS_PER_TILE)])
    plsc.subcore_barrier()

    def body(ci, _):
        off = base + ci * SCAT_CHUNK
        pltpu.sync_copy(dst_hbm.at[pl.ds(off, SCAT_CHUNK)], idx_v)
        pltpu.sync_copy(msg_hbm.at[pl.ds(off, SCAT_CHUNK)], msg_v)
        pltpu.sync_copy(msg_v, accum_sh.at[idx_v], add=True)
        return ()

    lax.fori_loop(0, (N_EDGES // NW) // SCAT_CHUNK, body, ())
    plsc.subcore_barrier()
    pltpu.sync_copy(accum_sh.at[pl.ds(row0, ROWS_PER_TILE)],
                    out_hbm.at[pl.ds(c * N_PAD + row0, ROWS_PER_TILE)])


def _sc_scatter(msg, dst, zeros):
    k = pl.kernel(
        _scatter_body,
        out_type=jax.ShapeDtypeStruct((NC * N_PAD, HIDDEN), jnp.float32),
        mesh=_MESH,
        compiler_params=pltpu.CompilerParams(use_tc_tiling_on_sc=False),
        scratch_types=[
            pltpu.VMEM((SCAT_CHUNK,), jnp.int32),
            pltpu.VMEM((SCAT_CHUNK, HIDDEN), jnp.float32),
            pltpu.VMEM_SHARED((N_PAD, HIDDEN), jnp.float32),
            pltpu.SemaphoreType.DMA,
        ],
    )
    return k(msg, dst, zeros)


EDGE_TILE = 2000


def _mlp_msg_body(d_in, ea_ref, xs_ref, xd_ref, wa_ref, ws_ref, wd_ref,
                  b0_ref, wmid_ref, bmid_ref, wlast_ref, blast_ref, out_ref):
    ea = ea_ref[...]
    xs = xs_ref[...]
    xd = xd_ref[...]
    f32 = jnp.float32
    h = (jnp.dot(ea, wa_ref[...], preferred_element_type=f32)
         + jnp.dot(xs, ws_ref[...], preferred_element_type=f32)
         + jnp.dot(xd, wd_ref[...], preferred_element_type=f32)
         + b0_ref[...])
    h = jnp.maximum(h, 0.0)
    for kk in range(14):
        h = jnp.dot(h, wmid_ref[kk], preferred_element_type=f32) + bmid_ref[kk]
        h = jnp.maximum(h, 0.0)
    w = jnp.dot(h, wlast_ref[...], preferred_element_type=f32) + blast_ref[...]
    msg = jnp.zeros((EDGE_TILE, HIDDEN), dtype=f32)
    for i in range(d_in):
        msg = msg + xs[:, i:i + 1] * w[:, i * HIDDEN:(i + 1) * HIDDEN]
    out_ref[...] = msg


def _tc_mlp_msg(d_in, ea, xs, xd, wa, ws, wd, b0, wmid, bmid, wlast, blast):
    grid = N_EDGES // EDGE_TILE
    rep = lambda s: pl.BlockSpec(s, lambda i: tuple(0 for _ in s))
    k = pl.pallas_call(
        functools.partial(_mlp_msg_body, d_in),
        grid=(grid,),
        in_specs=[
            pl.BlockSpec((EDGE_TILE, 2), lambda i: (i, 0)),
            pl.BlockSpec((EDGE_TILE, HIDDEN), lambda i: (i, 0)),
            pl.BlockSpec((EDGE_TILE, HIDDEN), lambda i: (i, 0)),
            rep((2, EDGE_W)),
            rep((HIDDEN, EDGE_W)),
            rep((HIDDEN, EDGE_W)),
            rep((1, EDGE_W)),
            rep((14, EDGE_W, EDGE_W)),
            rep((14, 1, EDGE_W)),
            rep((EDGE_W, d_in * HIDDEN)),
            rep((1, d_in * HIDDEN)),
        ],
        out_specs=pl.BlockSpec((EDGE_TILE, HIDDEN), lambda i: (i, 0)),
        out_shape=jax.ShapeDtypeStruct((N_EDGES, HIDDEN), jnp.float32),
    )
    return k(ea, xs, xd, wa, ws, wd, b0, wmid, bmid, wlast, blast)


def _update_body(residual, x_ref, p_ref, rw_ref, b_ref, out_ref):
    x = x_ref[...]
    aggr = p_ref[0:N_PAD] + p_ref[N_PAD:2 * N_PAD]
    out = jnp.dot(x, rw_ref[...], preferred_element_type=jnp.float32)
    out = out + aggr + b_ref[...]
    if residual:
        out = out + x
    out_ref[...] = jnp.where(out >= 0.0, out, 0.2 * out)


def _tc_update(residual, x, partials, root_w, bias):
    k = pl.pallas_call(
        functools.partial(_update_body, residual),
        out_shape=jax.ShapeDtypeStruct((N_PAD, HIDDEN), jnp.float32),
    )
    return k(x, partials, root_w, bias)


def _readout_body(x_ref, bi_ref, pw_ref, pb_ref, out_ref):
    x = x_ref[...]
    bi = bi_ref[...]
    gids = lax.broadcasted_iota(jnp.int32, (N_GRAPHS, N_PAD), 0)
    onehot = (bi == gids).astype(jnp.float32)
    s = jnp.dot(onehot, x, preferred_element_type=jnp.float32)
    cnt = jnp.sum(onehot, axis=1, keepdims=True)
    mean = s / jnp.maximum(cnt, 1.0)
    neg = jnp.float32(-jnp.inf)
    mxs = []
    for g in range(N_GRAPHS):
        mask = (bi[0][:, None] == g)
        mxs.append(jnp.max(jnp.where(mask, x, neg), axis=0, keepdims=True))
    mx = jnp.concatenate(mxs, axis=0)
    cat = jnp.concatenate([s, mean, mx], axis=1)
    out = jnp.dot(cat, pw_ref[...], preferred_element_type=jnp.float32)
    out_ref[...] = out + pb_ref[...]


def _tc_readout(x, batch_index, proj_w, proj_b):
    k = pl.pallas_call(
        _readout_body,
        out_shape=jax.ShapeDtypeStruct((N_GRAPHS, 1), jnp.float32),
    )
    bi = jnp.pad(batch_index, (0, N_PAD - N_NODES), constant_values=-1)
    return k(x, bi.reshape(1, N_PAD), proj_w,
             proj_b.reshape(1, 1))


def kernel(pos, edge_index, edge_attr, batch_index, params):
    f32 = jnp.float32
    idx_all = edge_index.reshape(2 * N_EDGES).astype(jnp.int32)
    dst = edge_index[1].astype(jnp.int32)
    zeros = jnp.zeros((N_PAD, HIDDEN), dtype=f32)

    x = jnp.pad(pos.astype(f32), ((0, N_PAD - N_NODES), (0, HIDDEN - 2)))
    for l, layer in enumerate(params["layers"]):
        d_in = 2 if l == 0 else HIDDEN
        mlp = layer["edge_mlp"]
        w0, b0 = mlp[0]
        wa = w0[:2]
        ws = w0[2:2 + d_in]
        wd = w0[2 + d_in:2 + 2 * d_in]
        root_w = layer["root_W"]
        if d_in < HIDDEN:
            padr = ((0, HIDDEN - d_in), (0, 0))
            ws = jnp.pad(ws, padr)
            wd = jnp.pad(wd, padr)
            root_w = jnp.pad(root_w, padr)
        wmid = jnp.stack([mlp[kk][0] for kk in range(1, 15)])
        bmid = jnp.stack([mlp[kk][1] for kk in range(1, 15)])[:, None, :]
        wlast, blast = mlp[15]

        g = _sc_gather(x, idx_all)
        xs = g[:N_EDGES]
        xd = g[N_EDGES:]
        msg = _tc_mlp_msg(d_in, edge_attr, xs, xd, wa, ws, wd,
                          b0.reshape(1, EDGE_W), wmid, bmid, wlast,
                          blast.reshape(1, d_in * HIDDEN))
        partials = _sc_scatter(msg, dst, zeros)
        x = _tc_update(l > 0, x, partials, root_w,
                       layer["bias"].reshape(1, HIDDEN))

    out = _tc_readout(x, batch_index.astype(jnp.int32),
                      params["proj_W"], params["proj_b"])
    return out.reshape(N_GRAPHS)


# speedup vs baseline: 1.6076x; 1.0457x over previous
import functools

import jax
import jax.numpy as jnp
from jax import lax
from jax.experimental import pallas as pl
from jax.experimental.pallas import tpu as pltpu
from jax.experimental.pallas import tpu_sc as plsc

N_NODES = 10000
N_PAD = 10240
N_EDGES = 160000
N_GRAPHS = 16
HIDDEN = 16
EDGE_W = 64

NC = 2
NS = 16
NW = NC * NS

_MESH = plsc.VectorSubcoreMesh(core_axis_name="c", subcore_axis_name="s")

GATHER_CHUNK = 2000


def _gather_body(x_hbm, idx_hbm, out_hbm, idx_v, rows_v, x_sh, sem):
    wid = lax.axis_index("s") * NC + lax.axis_index("c")
    s = lax.axis_index("s")
    row0 = s * ROWS_PER_TILE
    pltpu.sync_copy(x_hbm.at[pl.ds(row0, ROWS_PER_TILE)],
                    x_sh.at[pl.ds(row0, ROWS_PER_TILE)])
    plsc.subcore_barrier()
    base = wid * ((2 * N_EDGES) // NW)

    def body(ci, _):
        off = base + ci * GATHER_CHUNK
        pltpu.sync_copy(idx_hbm.at[pl.ds(off, GATHER_CHUNK)], idx_v)
        pltpu.async_copy(x_sh.at[idx_v], rows_v, sem).wait()
        pltpu.sync_copy(rows_v, out_hbm.at[pl.ds(off, GATHER_CHUNK)])
        return ()

    lax.fori_loop(0, (2 * N_EDGES) // NW // GATHER_CHUNK, body, ())


def _sc_gather(x, idx_all):
    k = pl.kernel(
        _gather_body,
        out_type=jax.ShapeDtypeStruct((2 * N_EDGES, HIDDEN), jnp.float32),
        mesh=_MESH,
        compiler_params=pltpu.CompilerParams(use_tc_tiling_on_sc=False),
        scratch_types=[
            pltpu.VMEM((GATHER_CHUNK,), jnp.int32),
            pltpu.VMEM((GATHER_CHUNK, HIDDEN), jnp.float32),
            pltpu.VMEM_SHARED((N_PAD, HIDDEN), jnp.float32),
            pltpu.SemaphoreType.DMA,
        ],
    )
    return k(x, idx_all)


SCAT_CHUNK = 200
ROWS_PER_TILE = N_PAD // NS


def _scatter_body(msg_hbm, dst_hbm, zeros_hbm, out_hbm, idx_v, msg_v,
                  accum_sh, sem):
    c = lax.axis_index("c")
    s = lax.axis_index("s")
    wid = s * NC + c
    base = wid * (N_EDGES // NW)
    row0 = s * ROWS_PER_TILE
    pltpu.sync_copy(zeros_hbm.at[pl.ds(row0, ROWS_PER_TILE)],
                    accum_sh.at[pl.ds(row0, ROWS_PER_TILE)])
    plsc.subcore_barrier()

    def body(ci, _):
        off = base + ci * SCAT_CHUNK
        pltpu.sync_copy(dst_hbm.at[pl.ds(off, SCAT_CHUNK)], idx_v)
        pltpu.sync_copy(msg_hbm.at[pl.ds(off, SCAT_CHUNK)], msg_v)
        pltpu.sync_copy(msg_v, accum_sh.at[idx_v], add=True)
        return ()

    lax.fori_loop(0, (N_EDGES // NW) // SCAT_CHUNK, body, ())
    plsc.subcore_barrier()
    pltpu.sync_copy(accum_sh.at[pl.ds(row0, ROWS_PER_TILE)],
                    out_hbm.at[pl.ds(c * N_PAD + row0, ROWS_PER_TILE)])


def _sc_scatter(msg, dst, zeros):
    k = pl.kernel(
        _scatter_body,
        out_type=jax.ShapeDtypeStruct((NC * N_PAD, HIDDEN), jnp.float32),
        mesh=_MESH,
        compiler_params=pltpu.CompilerParams(use_tc_tiling_on_sc=False),
        scratch_types=[
            pltpu.VMEM((SCAT_CHUNK,), jnp.int32),
            pltpu.VMEM((SCAT_CHUNK, HIDDEN), jnp.float32),
            pltpu.VMEM_SHARED((N_PAD, HIDDEN), jnp.float32),
            pltpu.SemaphoreType.DMA,
        ],
    )
    return k(msg, dst, zeros)


EDGE_TILE = 2000


def _mlp_msg_body(d_in, ea_ref, xs_ref, xd_ref, wa_ref, ws_ref, wd_ref,
                  b0_ref, wmid_ref, bmid_ref, wlast_ref, blast_ref, out_ref):
    bf16 = jnp.bfloat16
    f32 = jnp.float32
    ea = ea_ref[...].astype(bf16)
    xs = xs_ref[...]
    xd = xd_ref[...]
    h = (jnp.dot(ea, wa_ref[...], preferred_element_type=f32)
         + jnp.dot(xs.astype(bf16), ws_ref[...], preferred_element_type=f32)
         + jnp.dot(xd.astype(bf16), wd_ref[...], preferred_element_type=f32)
         + b0_ref[...])
    h = jnp.maximum(h, 0.0)
    for kk in range(14):
        h = jnp.dot(h.astype(bf16), wmid_ref[kk],
                    preferred_element_type=f32) + bmid_ref[kk]
        h = jnp.maximum(h, 0.0)
    w = (jnp.dot(h.astype(bf16), wlast_ref[...], preferred_element_type=f32)
         + blast_ref[...])
    msg = jnp.zeros((EDGE_TILE, HIDDEN), dtype=f32)
    for i in range(d_in):
        msg = msg + xs[:, i:i + 1] * w[:, i * HIDDEN:(i + 1) * HIDDEN]
    out_ref[...] = msg


def _tc_mlp_msg(d_in, ea, g, wa, ws, wd, b0, wmid, bmid, wlast, blast):
    grid = N_EDGES // EDGE_TILE
    rep = lambda s: pl.BlockSpec(s, lambda i: tuple(0 for _ in s))
    k = pl.pallas_call(
        functools.partial(_mlp_msg_body, d_in),
        grid=(grid,),
        in_specs=[
            pl.BlockSpec((EDGE_TILE, 2), lambda i: (i, 0)),
            pl.BlockSpec((EDGE_TILE, HIDDEN), lambda i: (i, 0)),
            pl.BlockSpec((EDGE_TILE, HIDDEN),
                         lambda i: (i + N_EDGES // EDGE_TILE, 0)),
            rep((2, EDGE_W)),
            rep((HIDDEN, EDGE_W)),
            rep((HIDDEN, EDGE_W)),
            rep((1, EDGE_W)),
            rep((14, EDGE_W, EDGE_W)),
            rep((14, 1, EDGE_W)),
            rep((EDGE_W, d_in * HIDDEN)),
            rep((1, d_in * HIDDEN)),
        ],
        out_specs=pl.BlockSpec((EDGE_TILE, HIDDEN), lambda i: (i, 0)),
        out_shape=jax.ShapeDtypeStruct((N_EDGES, HIDDEN), jnp.float32),
    )
    return k(ea, g, g, wa, ws, wd, b0, wmid, bmid, wlast, blast)


def _update_body(residual, x_ref, p_ref, rw_ref, b_ref, out_ref):
    x = x_ref[...]
    aggr = p_ref[0:N_PAD] + p_ref[N_PAD:2 * N_PAD]
    out = jnp.dot(x, rw_ref[...], preferred_element_type=jnp.float32)
    out = out + aggr + b_ref[...]
    if residual:
        out = out + x
    out_ref[...] = jnp.where(out >= 0.0, out, 0.2 * out)


def _tc_update(residual, x, partials, root_w, bias):
    k = pl.pallas_call(
        functools.partial(_update_body, residual),
        out_shape=jax.ShapeDtypeStruct((N_PAD, HIDDEN), jnp.float32),
    )
    return k(x, partials, root_w, bias)


def _readout_body(x_ref, bi_ref, pw_ref, pb_ref, out_ref):
    x = x_ref[...]
    bi = bi_ref[...]
    gids = lax.broadcasted_iota(jnp.int32, (N_GRAPHS, N_PAD), 0)
    onehot = (bi == gids).astype(jnp.float32)
    s = jnp.dot(onehot, x, preferred_element_type=jnp.float32)
    cnt = jnp.sum(onehot, axis=1, keepdims=True)
    mean = s / jnp.maximum(cnt, 1.0)
    neg = jnp.float32(-jnp.inf)
    mxs = []
    for g in range(N_GRAPHS):
        mask = (bi[0][:, None] == g)
        mxs.append(jnp.max(jnp.where(mask, x, neg), axis=0, keepdims=True))
    mx = jnp.concatenate(mxs, axis=0)
    cat = jnp.concatenate([s, mean, mx], axis=1)
    out = jnp.dot(cat, pw_ref[...], preferred_element_type=jnp.float32)
    out_ref[...] = out + pb_ref[...]


def _tc_readout(x, batch_index, proj_w, proj_b):
    k = pl.pallas_call(
        _readout_body,
        out_shape=jax.ShapeDtypeStruct((N_GRAPHS, 1), jnp.float32),
    )
    bi = jnp.pad(batch_index, (0, N_PAD - N_NODES), constant_values=-1)
    return k(x, bi.reshape(1, N_PAD), proj_w,
             proj_b.reshape(1, 1))


def kernel(pos, edge_index, edge_attr, batch_index, params):
    f32 = jnp.float32
    idx_all = edge_index.reshape(2 * N_EDGES).astype(jnp.int32)
    dst = edge_index[1].astype(jnp.int32)
    zeros = jnp.zeros((N_PAD, HIDDEN), dtype=f32)

    x = jnp.pad(pos.astype(f32), ((0, N_PAD - N_NODES), (0, HIDDEN - 2)))
    for l, layer in enumerate(params["layers"]):
        d_in = 2 if l == 0 else HIDDEN
        mlp = layer["edge_mlp"]
        w0, b0 = mlp[0]
        wa = w0[:2]
        ws = w0[2:2 + d_in]
        wd = w0[2 + d_in:2 + 2 * d_in]
        root_w = layer["root_W"]
        if d_in < HIDDEN:
            padr = ((0, HIDDEN - d_in), (0, 0))
            ws = jnp.pad(ws, padr)
            wd = jnp.pad(wd, padr)
            root_w = jnp.pad(root_w, padr)
        wmid = jnp.stack([mlp[kk][0] for kk in range(1, 15)])
        bmid = jnp.stack([mlp[kk][1] for kk in range(1, 15)])[:, None, :]
        wlast, blast = mlp[15]

        g = _sc_gather(x, idx_all)
        msg = _tc_mlp_msg(d_in, edge_attr, g, wa.astype(jnp.bfloat16),
                          ws.astype(jnp.bfloat16), wd.astype(jnp.bfloat16),
                          b0.reshape(1, EDGE_W), wmid.astype(jnp.bfloat16),
                          bmid, wlast.astype(jnp.bfloat16),
                          blast.reshape(1, d_in * HIDDEN))
        partials = _sc_scatter(msg, dst, zeros)
        x = _tc_update(l > 0, x, partials, root_w,
                       layer["bias"].reshape(1, HIDDEN))

    out = _tc_readout(x, batch_index.astype(jnp.int32),
                      params["proj_W"], params["proj_b"])
    return out.reshape(N_GRAPHS)


# speedup vs baseline: 2.9582x; 1.8402x over previous
import functools

import jax
import jax.numpy as jnp
from jax import lax
from jax.experimental import pallas as pl
from jax.experimental.pallas import tpu as pltpu
from jax.experimental.pallas import tpu_sc as plsc

N_NODES = 10000
N_PAD = 10240
N_EDGES = 160000
N_GRAPHS = 16
HIDDEN = 16
LANES = 128
EDGE_W = 64

NC = 2
NS = 16
NW = NC * NS

_MESH = plsc.VectorSubcoreMesh(core_axis_name="c", subcore_axis_name="s")
_SC_PARAMS = pltpu.CompilerParams(use_tc_tiling_on_sc=False)

GATHER_CHUNK = 2000
ROWS_PER_TILE = N_PAD // NS


def _gather_body(x_hbm, idx_hbm, out_hbm, idx_v, rows_v, x_sh, sem):
    wid = lax.axis_index("s") * NC + lax.axis_index("c")
    s = lax.axis_index("s")
    row0 = s * ROWS_PER_TILE
    pltpu.sync_copy(x_hbm.at[pl.ds(row0, ROWS_PER_TILE), pl.ds(0, HIDDEN)],
                    x_sh.at[pl.ds(row0, ROWS_PER_TILE)])
    plsc.subcore_barrier()
    base = wid * ((2 * N_EDGES) // NW)

    def body(ci, _):
        off = base + ci * GATHER_CHUNK
        pltpu.sync_copy(idx_hbm.at[pl.ds(off, GATHER_CHUNK)], idx_v)
        pltpu.async_copy(x_sh.at[idx_v], rows_v, sem).wait()
        pltpu.sync_copy(rows_v,
                        out_hbm.at[pl.ds(off, GATHER_CHUNK), pl.ds(0, HIDDEN)])
        return ()

    lax.fori_loop(0, (2 * N_EDGES) // NW // GATHER_CHUNK, body, ())


def _sc_gather(x, idx_all):
    k = pl.kernel(
        _gather_body,
        out_type=jax.ShapeDtypeStruct((2 * N_EDGES, LANES), jnp.float32),
        mesh=_MESH,
        compiler_params=_SC_PARAMS,
        scratch_types=[
            pltpu.VMEM((GATHER_CHUNK,), jnp.int32),
            pltpu.VMEM((GATHER_CHUNK, HIDDEN), jnp.float32),
            pltpu.VMEM_SHARED((N_PAD, HIDDEN), jnp.float32),
            pltpu.SemaphoreType.DMA,
        ],
    )
    return k(x, idx_all)


SCAT_CHUNK = 200


def _scatter_body(msg_hbm, dst_hbm, zeros_hbm, out_hbm, idx_v, msg_v,
                  accum_sh, sem):
    c = lax.axis_index("c")
    s = lax.axis_index("s")
    wid = s * NC + c
    base = wid * (N_EDGES // NW)
    row0 = s * ROWS_PER_TILE
    pltpu.sync_copy(zeros_hbm.at[pl.ds(row0, ROWS_PER_TILE), pl.ds(0, HIDDEN)],
                    accum_sh.at[pl.ds(row0, ROWS_PER_TILE)])
    plsc.subcore_barrier()

    def body(ci, _):
        off = base + ci * SCAT_CHUNK
        pltpu.sync_copy(dst_hbm.at[pl.ds(off, SCAT_CHUNK)], idx_v)
        pltpu.sync_copy(msg_hbm.at[pl.ds(off, SCAT_CHUNK), pl.ds(0, HIDDEN)],
                        msg_v)
        pltpu.sync_copy(msg_v, accum_sh.at[idx_v], add=True)
        return ()

    lax.fori_loop(0, (N_EDGES // NW) // SCAT_CHUNK, body, ())
    plsc.subcore_barrier()
    pltpu.sync_copy(accum_sh.at[pl.ds(row0, ROWS_PER_TILE)],
                    out_hbm.at[pl.ds(c * N_PAD + row0, ROWS_PER_TILE),
                               pl.ds(0, HIDDEN)])


def _sc_scatter(msg, dst, zeros):
    k = pl.kernel(
        _scatter_body,
        out_type=jax.ShapeDtypeStruct((NC * N_PAD, LANES), jnp.float32),
        mesh=_MESH,
        compiler_params=_SC_PARAMS,
        scratch_types=[
            pltpu.VMEM((SCAT_CHUNK,), jnp.int32),
            pltpu.VMEM((SCAT_CHUNK, HIDDEN), jnp.float32),
            pltpu.VMEM_SHARED((N_PAD, HIDDEN), jnp.float32),
            pltpu.SemaphoreType.DMA,
        ],
    )
    return k(msg, dst, zeros)


EDGE_TILE = 2000


def _mlp_msg_body(ea_ref, xs_ref, xd_ref, wa_ref, ws_ref, wd_ref,
                  b0_ref, wmid_ref, bmid_ref, wlast_ref, blast_ref,
                  rmat_ref, smat_ref, out_ref):
    bf16 = jnp.bfloat16
    f32 = jnp.float32
    ea = ea_ref[...].astype(bf16)
    xs = xs_ref[:, :HIDDEN]
    xd = xd_ref[:, :HIDDEN]
    h = (jnp.dot(ea, wa_ref[...], preferred_element_type=f32)
         + jnp.dot(xs.astype(bf16), ws_ref[...], preferred_element_type=f32)
         + jnp.dot(xd.astype(bf16), wd_ref[...], preferred_element_type=f32)
         + b0_ref[...])
    h = jnp.maximum(h, 0.0)
    for kk in range(14):
        h = jnp.dot(h.astype(bf16), wmid_ref[kk],
                    preferred_element_type=f32) + bmid_ref[kk]
        h = jnp.maximum(h, 0.0)
    w = (jnp.dot(h.astype(bf16), wlast_ref[...], preferred_element_type=f32)
         + blast_ref[...])
    xsrep = jnp.dot(xs, rmat_ref[...], preferred_element_type=f32)
    msg = jnp.dot(xsrep * w, smat_ref[...], preferred_element_type=f32)
    out_ref[...] = jnp.pad(msg, ((0, 0), (0, LANES - HIDDEN)))


def _tc_mlp_msg(d_in, ea, g, wa, ws, wd, b0, wmid, bmid, wlast, blast,
                rmat, smat):
    grid = N_EDGES // EDGE_TILE
    rep = lambda s: pl.BlockSpec(s, lambda i: tuple(0 for _ in s))
    k = pl.pallas_call(
        _mlp_msg_body,
        grid=(grid,),
        in_specs=[
            pl.BlockSpec((EDGE_TILE, 2), lambda i: (i, 0)),
            pl.BlockSpec((EDGE_TILE, LANES), lambda i: (i, 0)),
            pl.BlockSpec((EDGE_TILE, LANES),
                         lambda i: (i + N_EDGES // EDGE_TILE, 0)),
            rep((2, EDGE_W)),
            rep((HIDDEN, EDGE_W)),
            rep((HIDDEN, EDGE_W)),
            rep((1, EDGE_W)),
            rep((14, EDGE_W, EDGE_W)),
            rep((14, 1, EDGE_W)),
            rep((EDGE_W, d_in * HIDDEN)),
            rep((1, d_in * HIDDEN)),
            rep((HIDDEN, d_in * HIDDEN)),
            rep((d_in * HIDDEN, HIDDEN)),
        ],
        out_specs=pl.BlockSpec((EDGE_TILE, LANES), lambda i: (i, 0)),
        out_shape=jax.ShapeDtypeStruct((N_EDGES, LANES), jnp.float32),
    )
    return k(ea, g, g, wa, ws, wd, b0, wmid, bmid, wlast, blast, rmat, smat)


def _update_body(residual, x_ref, p_ref, rw_ref, b_ref, out_ref):
    x = x_ref[:, :HIDDEN]
    aggr = p_ref[0:N_PAD, :HIDDEN] + p_ref[N_PAD:2 * N_PAD, :HIDDEN]
    out = jnp.dot(x, rw_ref[...], preferred_element_type=jnp.float32)
    out = out + aggr + b_ref[...]
    if residual:
        out = out + x
    out = jnp.where(out >= 0.0, out, 0.2 * out)
    out_ref[...] = jnp.pad(out, ((0, 0), (0, LANES - HIDDEN)))


def _tc_update(residual, x, partials, root_w, bias):
    k = pl.pallas_call(
        functools.partial(_update_body, residual),
        grid=(1,),
        in_specs=[
            pl.BlockSpec((N_PAD, LANES), lambda i: (0, 0)),
            pl.BlockSpec((2 * N_PAD, LANES), lambda i: (0, 0)),
            pl.BlockSpec((HIDDEN, HIDDEN), lambda i: (0, 0)),
            pl.BlockSpec((1, HIDDEN), lambda i: (0, 0)),
        ],
        out_specs=pl.BlockSpec((N_PAD, LANES), lambda i: (0, 0)),
        out_shape=jax.ShapeDtypeStruct((N_PAD, LANES), jnp.float32),
    )
    return k(x, partials, root_w, bias)


def _readout_body(x_ref, bi_ref, pw_ref, pb_ref, out_ref):
    x = x_ref[:, :HIDDEN]
    bi = bi_ref[...]
    gids = lax.broadcasted_iota(jnp.int32, (N_GRAPHS, N_PAD), 0)
    onehot = (bi == gids).astype(jnp.float32)
    s = jnp.dot(onehot, x, preferred_element_type=jnp.float32)
    cnt = jnp.sum(onehot, axis=1, keepdims=True)
    mean = s / jnp.maximum(cnt, 1.0)
    neg = jnp.float32(-jnp.inf)
    mxs = []
    for g in range(N_GRAPHS):
        mask = (bi[0][:, None] == g)
        mxs.append(jnp.max(jnp.where(mask, x, neg), axis=0, keepdims=True))
    mx = jnp.concatenate(mxs, axis=0)
    cat = jnp.concatenate([s, mean, mx], axis=1)
    out = jnp.dot(cat, pw_ref[...], preferred_element_type=jnp.float32)
    out_ref[...] = out + pb_ref[...]


def _tc_readout(x, batch_index, proj_w, proj_b):
    k = pl.pallas_call(
        _readout_body,
        grid=(1,),
        in_specs=[
            pl.BlockSpec((N_PAD, LANES), lambda i: (0, 0)),
            pl.BlockSpec((1, N_PAD), lambda i: (0, 0)),
            pl.BlockSpec((3 * HIDDEN, 1), lambda i: (0, 0)),
            pl.BlockSpec((1, 1), lambda i: (0, 0)),
        ],
        out_specs=pl.BlockSpec((N_GRAPHS, 1), lambda i: (0, 0)),
        out_shape=jax.ShapeDtypeStruct((N_GRAPHS, 1), jnp.float32),
    )
    bi = jnp.pad(batch_index, (0, N_PAD - N_NODES), constant_values=-1)
    return k(x, bi.reshape(1, N_PAD), proj_w, proj_b.reshape(1, 1))


def kernel(pos, edge_index, edge_attr, batch_index, params):
    f32 = jnp.float32
    idx_all = edge_index.reshape(2 * N_EDGES).astype(jnp.int32)
    dst = edge_index[1].astype(jnp.int32)
    zeros = jnp.zeros((N_PAD, LANES), dtype=f32)

    x = jnp.zeros((N_PAD, LANES), dtype=f32)
    x = x.at[:N_NODES, :2].set(pos.astype(f32))
    for l, layer in enumerate(params["layers"]):
        d_in = 2 if l == 0 else HIDDEN
        mlp = layer["edge_mlp"]
        w0, b0 = mlp[0]
        wa = w0[:2]
        ws = w0[2:2 + d_in]
        wd = w0[2 + d_in:2 + 2 * d_in]
        root_w = layer["root_W"]
        if d_in < HIDDEN:
            padr = ((0, HIDDEN - d_in), (0, 0))
            ws = jnp.pad(ws, padr)
            wd = jnp.pad(wd, padr)
            root_w = jnp.pad(root_w, padr)
        wmid = jnp.stack([mlp[kk][0] for kk in range(1, 15)])
        bmid = jnp.stack([mlp[kk][1] for kk in range(1, 15)])[:, None, :]
        wlast, blast = mlp[15]
        eye = jnp.eye(HIDDEN, dtype=f32)
        rmat = jnp.kron(eye[:, :d_in], jnp.ones((1, HIDDEN), dtype=f32))
        smat = jnp.kron(jnp.ones((d_in, 1), dtype=f32), eye)

        g = _sc_gather(x, idx_all)
        msg = _tc_mlp_msg(d_in, edge_attr, g, wa.astype(jnp.bfloat16),
                          ws.astype(jnp.bfloat16), wd.astype(jnp.bfloat16),
                          b0.reshape(1, EDGE_W), wmid.astype(jnp.bfloat16),
                          bmid, wlast.astype(jnp.bfloat16),
                          blast.reshape(1, d_in * HIDDEN), rmat, smat)
        partials = _sc_scatter(msg, dst, zeros)
        x = _tc_update(l > 0, x, partials, root_w,
                       layer["bias"].reshape(1, HIDDEN))

    out = _tc_readout(x, batch_index.astype(jnp.int32),
                      params["proj_W"], params["proj_b"])
    return out.reshape(N_GRAPHS)
